# Initial kernel scaffold; baseline (speedup 1.0000x reference)
import functools

import jax
import jax.numpy as jnp
from jax import lax
from jax.experimental import pallas as pl
from jax.experimental.pallas import tpu as pltpu
from jax.experimental.pallas import tpu_sc as plsc

N = 10000
D = 128
E = 320000
K = 3

NC = 2
NS = 16
NW = NC * NS

CH = 128
GRP = 4
G = 20
T = G * GRP
E_PAD = NW * T * CH
NP = 10240
RPT = NP // NS
TRASH = N

_mesh = plsc.VectorSubcoreMesh(core_axis_name="c", subcore_axis_name="s")


def _hop_body(cur, srcg, dstg, zer, out, acc, sidx, didx, rows, sem):
    cid = lax.axis_index("c")
    sid = lax.axis_index("s")
    wid = sid * NC + cid
    pltpu.sync_copy(zer.at[pl.ds(sid * RPT, RPT)],
                    acc.at[pl.ds(sid * RPT, RPT)])
    plsc.subcore_barrier()

    def group(g, carry):
        gi = wid * G + g
        pltpu.sync_copy(srcg.at[gi], sidx)
        pltpu.sync_copy(dstg.at[gi], didx)
        descs = [pltpu.async_copy(cur.at[sidx.at[j]], rows.at[j], sem)
                 for j in range(GRP)]
        for d in descs:
            d.wait()
        descs = [pltpu.async_copy(rows.at[j], acc.at[didx.at[j]], sem,
                                  add=True)
                 for j in range(GRP)]
        for d in descs:
            d.wait()
        return carry

    lax.fori_loop(0, G, group, 0)
    plsc.subcore_barrier()
    pltpu.sync_copy(acc.at[pl.ds(sid * RPT, RPT)],
                    out.at[cid, pl.ds(sid * RPT, RPT)])


_hop = functools.partial(
    pl.kernel,
    out_type=jax.ShapeDtypeStruct((NC, NP, D), jnp.float32),
    mesh=_mesh,
    scratch_types=[
        pltpu.VMEM_SHARED((NP, D), jnp.float32),
        pltpu.VMEM((GRP, CH), jnp.int32),
        pltpu.VMEM((GRP, CH), jnp.int32),
        pltpu.VMEM((GRP, CH, D), jnp.float32),
        pltpu.SemaphoreType.DMA,
    ],
)(_hop_body)


BLK = 1024


def _mm_body(x_ref, w_ref, o_ref):
    o_ref[...] = jnp.dot(x_ref[...], w_ref[...],
                         preferred_element_type=jnp.float32)


_mm = pl.pallas_call(
    _mm_body,
    grid=(NP // BLK,),
    in_specs=[
        pl.BlockSpec((BLK, D), lambda i: (i, 0)),
        pl.BlockSpec((D, D), lambda i: (0, 0)),
    ],
    out_specs=pl.BlockSpec((BLK, D), lambda i: (i, 0)),
    out_shape=jax.ShapeDtypeStruct((NP, D), jnp.float32),
)


def _comb_body(p_ref, o_ref):
    o_ref[...] = p_ref[0] + p_ref[1]


_comb = pl.pallas_call(
    _comb_body,
    grid=(NP // BLK,),
    in_specs=[pl.BlockSpec((NC, BLK, D), lambda i: (0, i, 0))],
    out_specs=pl.BlockSpec((BLK, D), lambda i: (i, 0)),
    out_shape=jax.ShapeDtypeStruct((NP, D), jnp.float32),
)


def _att_body(h0, h1, h2, h3, x_ref, w_ref, q_ref, b_ref, o_ref, *, relu):
    qa = q_ref[0:1, :]
    qb = q_ref[1:2, :]
    hs = [h0[...], h1[...], h2[...], h3[...]]
    t = jnp.sum(hs[0] * qa, axis=1, keepdims=True)
    ss = [t + jnp.sum(h * qb, axis=1, keepdims=True) for h in hs]
    ss = [jnp.where(s >= 0, s, 0.2 * s) for s in ss]
    m = jnp.maximum(jnp.maximum(ss[0], ss[1]), jnp.maximum(ss[2], ss[3]))
    es = [jnp.exp(s - m) for s in ss]
    den = es[0] + es[1] + es[2] + es[3]
    out = (es[0] * hs[0] + es[1] * hs[1] + es[2] * hs[2] + es[3] * hs[3])
    out = out / den
    out = out + jnp.dot(x_ref[...], w_ref[...],
                        preferred_element_type=jnp.float32) + b_ref[0:1, :]
    if relu:
        out = jnp.maximum(out, 0.0)
    o_ref[...] = out


def _att(relu):
    return pl.pallas_call(
        functools.partial(_att_body, relu=relu),
        grid=(NP // BLK,),
        in_specs=[
            pl.BlockSpec((BLK, D), lambda i: (i, 0)),
            pl.BlockSpec((BLK, D), lambda i: (i, 0)),
            pl.BlockSpec((BLK, D), lambda i: (i, 0)),
            pl.BlockSpec((BLK, D), lambda i: (i, 0)),
            pl.BlockSpec((BLK, D), lambda i: (i, 0)),
            pl.BlockSpec((D, D), lambda i: (0, 0)),
            pl.BlockSpec((2, D), lambda i: (0, 0)),
            pl.BlockSpec((1, D), lambda i: (0, 0)),
        ],
        out_specs=pl.BlockSpec((BLK, D), lambda i: (i, 0)),
        out_shape=jax.ShapeDtypeStruct((NP, D), jnp.float32),
    )


_att_relu = _att(True)
_att_plain = _att(False)


def kernel(x, edge_index, W1, Wres1, b1, q1, W2, Wres2, b2, q2):
    src = edge_index[0].astype(jnp.int32)
    dst = edge_index[1].astype(jnp.int32)
    srcg = jnp.concatenate(
        [src, jnp.zeros((E_PAD - E,), jnp.int32)]).reshape(NW * G, GRP, CH)
    dstg = jnp.concatenate(
        [dst, jnp.full((E_PAD - E,), TRASH, jnp.int32)]).reshape(NW * G, GRP, CH)
    xp = jnp.zeros((NP, D), jnp.float32).at[:N].set(x)
    zer = jnp.zeros((NP, D), jnp.float32)

    def layer(xin, W, Wres, b, q, relu):
        h0 = _mm(xin, W)
        cur = h0
        hs = [h0]
        for _ in range(K):
            p = _hop(cur, srcg, dstg, zer)
            cur = _comb(p)
            hs.append(cur)
        att = _att_relu if relu else _att_plain
        q2d = q.reshape(2, D)
        b2d = b.reshape(1, D)
        return att(hs[0], hs[1], hs[2], hs[3], xin, Wres, q2d, b2d)

    h = layer(xp, W1, Wres1, b1, q1, True)
    out = layer(h, W2, Wres2, b2, q2, False)
    return out[:N]


# speedup vs baseline: 2.8070x; 2.8070x over previous
import functools

import jax
import jax.numpy as jnp
from jax import lax
from jax.experimental import pallas as pl
from jax.experimental.pallas import tpu as pltpu
from jax.experimental.pallas import tpu_sc as plsc

N = 10000
D = 128
E = 320000
K = 3

NC = 2
NS = 16
NW = NC * NS

CH = 128
GRP = 2
G = 40
T = G * GRP
E_PAD = NW * T * CH
NP = 10240
RPT = NP // NS
TRASH = N

_mesh = plsc.VectorSubcoreMesh(core_axis_name="c", subcore_axis_name="s")


def _hop_body(cur, srcg, dstg, zer, out, acc, sidx, didx, rows, sem):
    cid = lax.axis_index("c")
    sid = lax.axis_index("s")
    wid = sid * NC + cid
    pltpu.sync_copy(zer.at[pl.ds(sid * RPT, RPT)],
                    acc.at[pl.ds(sid * RPT, RPT)])
    plsc.subcore_barrier()

    def group(g, carry):
        gi = wid * G + g
        pltpu.sync_copy(srcg.at[gi], sidx)
        pltpu.sync_copy(dstg.at[gi], didx)
        descs = [pltpu.async_copy(cur.at[sidx.at[j]], rows.at[j], sem)
                 for j in range(GRP)]
        for d in descs:
            d.wait()
        descs = [pltpu.async_copy(rows.at[j], acc.at[didx.at[j]], sem,
                                  add=True)
                 for j in range(GRP)]
        for d in descs:
            d.wait()
        return carry

    lax.fori_loop(0, G, group, 0)
    plsc.subcore_barrier()
    pltpu.sync_copy(acc.at[pl.ds(sid * RPT, RPT)],
                    out.at[cid, pl.ds(sid * RPT, RPT)])


_hop = functools.partial(
    pl.kernel,
    out_type=jax.ShapeDtypeStruct((NC, NP, D), jnp.float32),
    mesh=_mesh,
    scratch_types=[
        pltpu.VMEM_SHARED((NP, D), jnp.float32),
        pltpu.VMEM((GRP, CH), jnp.int32),
        pltpu.VMEM((GRP, CH), jnp.int32),
        pltpu.VMEM((GRP, CH, D), jnp.float32),
        pltpu.SemaphoreType.DMA,
    ],
)(_hop_body)


BLK = 1024


def _mm_body(x_ref, w_ref, o_ref):
    o_ref[...] = jnp.dot(x_ref[...], w_ref[...],
                         preferred_element_type=jnp.float32)


_mm = pl.pallas_call(
    _mm_body,
    grid=(NP // BLK,),
    in_specs=[
        pl.BlockSpec((BLK, D), lambda i: (i, 0)),
        pl.BlockSpec((D, D), lambda i: (0, 0)),
    ],
    out_specs=pl.BlockSpec((BLK, D), lambda i: (i, 0)),
    out_shape=jax.ShapeDtypeStruct((NP, D), jnp.float32),
)


def _comb_body(p_ref, o_ref):
    o_ref[...] = p_ref[0] + p_ref[1]


_comb = pl.pallas_call(
    _comb_body,
    grid=(NP // BLK,),
    in_specs=[pl.BlockSpec((NC, BLK, D), lambda i: (0, i, 0))],
    out_specs=pl.BlockSpec((BLK, D), lambda i: (i, 0)),
    out_shape=jax.ShapeDtypeStruct((NP, D), jnp.float32),
)


def _att_body(h0, h1, h2, h3, x_ref, w_ref, q_ref, b_ref, o_ref, *, relu):
    qa = q_ref[0:1, :]
    qb = q_ref[1:2, :]
    hs = [h0[...], h1[...], h2[...], h3[...]]
    t = jnp.sum(hs[0] * qa, axis=1, keepdims=True)
    ss = [t + jnp.sum(h * qb, axis=1, keepdims=True) for h in hs]
    ss = [jnp.where(s >= 0, s, 0.2 * s) for s in ss]
    m = jnp.maximum(jnp.maximum(ss[0], ss[1]), jnp.maximum(ss[2], ss[3]))
    es = [jnp.exp(s - m) for s in ss]
    den = es[0] + es[1] + es[2] + es[3]
    out = (es[0] * hs[0] + es[1] * hs[1] + es[2] * hs[2] + es[3] * hs[3])
    out = out / den
    out = out + jnp.dot(x_ref[...], w_ref[...],
                        preferred_element_type=jnp.float32) + b_ref[0:1, :]
    if relu:
        out = jnp.maximum(out, 0.0)
    o_ref[...] = out


def _att(relu):
    return pl.pallas_call(
        functools.partial(_att_body, relu=relu),
        grid=(NP // BLK,),
        in_specs=[
            pl.BlockSpec((BLK, D), lambda i: (i, 0)),
            pl.BlockSpec((BLK, D), lambda i: (i, 0)),
            pl.BlockSpec((BLK, D), lambda i: (i, 0)),
            pl.BlockSpec((BLK, D), lambda i: (i, 0)),
            pl.BlockSpec((BLK, D), lambda i: (i, 0)),
            pl.BlockSpec((D, D), lambda i: (0, 0)),
            pl.BlockSpec((2, D), lambda i: (0, 0)),
            pl.BlockSpec((1, D), lambda i: (0, 0)),
        ],
        out_specs=pl.BlockSpec((BLK, D), lambda i: (i, 0)),
        out_shape=jax.ShapeDtypeStruct((NP, D), jnp.float32),
    )


_att_relu = _att(True)
_att_plain = _att(False)


def kernel(x, edge_index, W1, Wres1, b1, q1, W2, Wres2, b2, q2):
    src = edge_index[0].astype(jnp.int32)
    dst = edge_index[1].astype(jnp.int32)
    srcg = jnp.concatenate(
        [src, jnp.zeros((E_PAD - E,), jnp.int32)]).reshape(NW * G, GRP, CH)
    dstg = jnp.concatenate(
        [dst, jnp.full((E_PAD - E,), TRASH, jnp.int32)]).reshape(NW * G, GRP, CH)
    xp = jnp.zeros((NP, D), jnp.float32).at[:N].set(x)
    zer = jnp.zeros((NP, D), jnp.float32)

    def layer(xin, W, Wres, b, q, relu):
        h0 = _mm(xin, W)
        cur = h0
        hs = [h0]
        for _ in range(K):
            p = _hop(cur, srcg, dstg, zer)
            cur = _comb(p)
            hs.append(cur)
        att = _att_relu if relu else _att_plain
        q2d = q.reshape(2, D)
        b2d = b.reshape(1, D)
        return att(hs[0], hs[1], hs[2], hs[3], xin, Wres, q2d, b2d)

    h = layer(xp, W1, Wres1, b1, q1, True)
    out = layer(h, W2, Wres2, b2, q2, False)
    return out[:N]


# speedup vs baseline: 3.0097x; 1.0722x over previous
import functools

import jax
import jax.numpy as jnp
from jax import lax
from jax.experimental import pallas as pl
from jax.experimental.pallas import tpu as pltpu
from jax.experimental.pallas import tpu_sc as plsc

N = 10000
D = 128
E = 320000
K = 3

NC = 2
NS = 16
NW = NC * NS

CH = 128
T = 80
T2 = T + 16
E_PAD = NW * T * CH
NP = 10240
RPT = NP // NS
TRASH = N

_mesh = plsc.VectorSubcoreMesh(core_axis_name="c", subcore_axis_name="s")


BC = 8
NB = T // BC


def _hop_body(cur, srcg, dstg, zer, out, acc, srcb, dstb, rows,
              sem_g, sem_i, sem_s):
    cid = lax.axis_index("c")
    sid = lax.axis_index("s")
    wid = sid * NC + cid
    row0 = wid * T2

    def idx_start(b, s):
        pltpu.async_copy(srcg.at[pl.ds(row0 + b * BC, BC)], srcb.at[s],
                         sem_i)
        pltpu.async_copy(dstg.at[pl.ds(row0 + b * BC, BC)], dstb.at[s],
                         sem_i)

    def idx_wait(b, s):
        pltpu.make_async_copy(srcg.at[pl.ds(row0 + b * BC, BC)],
                              srcb.at[s], sem_i).wait()
        pltpu.make_async_copy(dstg.at[pl.ds(row0 + b * BC, BC)],
                              dstb.at[s], sem_i).wait()

    def block(b, s):
        pltpu.sync_copy(srcg.at[pl.ds(row0 + b * BC, BC)], srcb.at[s])
        pltpu.sync_copy(dstg.at[pl.ds(row0 + b * BC, BC)], dstb.at[s])
        for jj in range(BC // 2):
            dg = [pltpu.async_copy(cur.at[srcb.at[s, 2 * jj + u]],
                                   rows.at[u], sem_g)
                  for u in range(2)]
            for d in dg:
                d.wait()
            for u in range(2):
                pltpu.async_copy(rows.at[u], acc.at[dstb.at[s, 2 * jj + u]],
                                 sem_s, add=True).wait()

    pltpu.sync_copy(zer.at[pl.ds(sid * RPT, RPT)],
                    acc.at[pl.ds(sid * RPT, RPT)])
    plsc.subcore_barrier()

    def pair(k, carry):
        block(2 * k, 0)
        block(2 * k + 1, 1)
        return carry

    lax.fori_loop(0, NB // 2, pair, 0)

    plsc.subcore_barrier()
    pltpu.sync_copy(acc.at[pl.ds(sid * RPT, RPT)],
                    out.at[cid, pl.ds(sid * RPT, RPT)])


_hop = functools.partial(
    pl.kernel,
    out_type=jax.ShapeDtypeStruct((NC, NP, D), jnp.float32),
    mesh=_mesh,
    scratch_types=[
        pltpu.VMEM_SHARED((NP, D), jnp.float32),
        pltpu.VMEM((2, BC, CH), jnp.int32),
        pltpu.VMEM((2, BC, CH), jnp.int32),
        pltpu.VMEM((2, CH, D), jnp.float32),
        pltpu.SemaphoreType.DMA,
        pltpu.SemaphoreType.DMA,
        pltpu.SemaphoreType.DMA,
    ],
)(_hop_body)


BLK = 1024


def _mm_body(x_ref, w_ref, o_ref):
    o_ref[...] = jnp.dot(x_ref[...], w_ref[...],
                         preferred_element_type=jnp.float32)


_mm = pl.pallas_call(
    _mm_body,
    grid=(NP // BLK,),
    in_specs=[
        pl.BlockSpec((BLK, D), lambda i: (i, 0)),
        pl.BlockSpec((D, D), lambda i: (0, 0)),
    ],
    out_specs=pl.BlockSpec((BLK, D), lambda i: (i, 0)),
    out_shape=jax.ShapeDtypeStruct((NP, D), jnp.float32),
)


def _comb_body(p_ref, o_ref):
    o_ref[...] = p_ref[0] + p_ref[1]


_comb = pl.pallas_call(
    _comb_body,
    grid=(NP // BLK,),
    in_specs=[pl.BlockSpec((NC, BLK, D), lambda i: (0, i, 0))],
    out_specs=pl.BlockSpec((BLK, D), lambda i: (i, 0)),
    out_shape=jax.ShapeDtypeStruct((NP, D), jnp.float32),
)


def _att_body(h0, h1, h2, h3, x_ref, w_ref, q_ref, b_ref, o_ref, *, relu):
    hs = [h0[...], h1[...], h2[...], h3[...]]
    rs = [jnp.dot(h, q_ref[...], preferred_element_type=jnp.float32)
          for h in hs]
    t = rs[0][:, 0:1]
    ss = [t + r[:, 1:2] for r in rs]
    ss = [jnp.where(s >= 0, s, 0.2 * s) for s in ss]
    m = jnp.maximum(jnp.maximum(ss[0], ss[1]), jnp.maximum(ss[2], ss[3]))
    es = [jnp.exp(s - m) for s in ss]
    den = es[0] + es[1] + es[2] + es[3]
    out = (es[0] * hs[0] + es[1] * hs[1] + es[2] * hs[2] + es[3] * hs[3])
    out = out / den
    out = out + jnp.dot(x_ref[...], w_ref[...],
                        preferred_element_type=jnp.float32) + b_ref[0:1, :]
    if relu:
        out = jnp.maximum(out, 0.0)
    o_ref[...] = out


def _att(relu):
    return pl.pallas_call(
        functools.partial(_att_body, relu=relu),
        grid=(NP // BLK,),
        in_specs=[
            pl.BlockSpec((BLK, D), lambda i: (i, 0)),
            pl.BlockSpec((BLK, D), lambda i: (i, 0)),
            pl.BlockSpec((BLK, D), lambda i: (i, 0)),
            pl.BlockSpec((BLK, D), lambda i: (i, 0)),
            pl.BlockSpec((BLK, D), lambda i: (i, 0)),
            pl.BlockSpec((D, D), lambda i: (0, 0)),
            pl.BlockSpec((D, 8), lambda i: (0, 0)),
            pl.BlockSpec((1, D), lambda i: (0, 0)),
        ],
        out_specs=pl.BlockSpec((BLK, D), lambda i: (i, 0)),
        out_shape=jax.ShapeDtypeStruct((NP, D), jnp.float32),
    )


_att_relu = _att(True)
_att_plain = _att(False)


def kernel(x, edge_index, W1, Wres1, b1, q1, W2, Wres2, b2, q2):
    src = edge_index[0].astype(jnp.int32)
    dst = edge_index[1].astype(jnp.int32)
    srcg = jnp.concatenate(
        [src, jnp.zeros((E_PAD - E,), jnp.int32)]).reshape(NW, T, CH)
    srcg = jnp.concatenate(
        [srcg, jnp.zeros((NW, T2 - T, CH), jnp.int32)],
        axis=1).reshape(NW * T2, CH)
    dstg = jnp.concatenate(
        [dst, jnp.full((E_PAD - E,), TRASH, jnp.int32)]).reshape(NW, T, CH)
    dstg = jnp.concatenate(
        [dstg, jnp.full((NW, T2 - T, CH), TRASH, jnp.int32)],
        axis=1).reshape(NW * T2, CH)
    xp = jnp.zeros((NP, D), jnp.float32).at[:N].set(x)
    zer = jnp.zeros((NP, D), jnp.float32)

    def layer(xin, W, Wres, b, q, relu):
        h0 = _mm(xin, W)
        cur = h0
        hs = [h0]
        for _ in range(K):
            p = _hop(cur, srcg, dstg, zer)
            cur = _comb(p)
            hs.append(cur)
        att = _att_relu if relu else _att_plain
        qm = jnp.zeros((D, 8), jnp.float32)
        qm = qm.at[:, 0].set(q[:D]).at[:, 1].set(q[D:])
        b2d = b.reshape(1, D)
        return att(hs[0], hs[1], hs[2], hs[3], xin, Wres, qm, b2d)

    h = layer(xp, W1, Wres1, b1, q1, True)
    out = layer(h, W2, Wres2, b2, q2, False)
    return out[:N]


# speedup vs baseline: 5.4069x; 1.7965x over previous
import functools

import jax
import jax.numpy as jnp
from jax import lax
from jax.experimental import pallas as pl
from jax.experimental.pallas import tpu as pltpu
from jax.experimental.pallas import tpu_sc as plsc

N = 10000
D = 128
HD = D // 2
E = 320000
K = 3

NC = 2
NS = 16
NW = NC * NS

CH = 128
GRP = 2
BC = 8
T = 80
T2 = T + 16
NB = T // BC
E_PAD = NW * T * CH
NP = 10240
RPT = NP // NS
TRASH = N

_mesh = plsc.VectorSubcoreMesh(core_axis_name="c", subcore_axis_name="s")


def _hop_body(cur2, srcg, dstg, zer, out, tab, acc, srcb, dstb, rows,
              sem_g, sem_s):
    cid = lax.axis_index("c")
    sid = lax.axis_index("s")
    wid = sid * NC + cid
    row0 = wid * T2
    rsl = pl.ds(sid * RPT, RPT)

    for p in range(2):
        pltpu.sync_copy(cur2.at[p, rsl], tab.at[rsl])
        pltpu.sync_copy(zer.at[rsl], acc.at[rsl])
        plsc.subcore_barrier()

        def block(b, s):
            pltpu.sync_copy(srcg.at[pl.ds(row0 + b * BC, BC)], srcb.at[s])
            pltpu.sync_copy(dstg.at[pl.ds(row0 + b * BC, BC)], dstb.at[s])
            for jj in range(BC // GRP):
                dg = [pltpu.async_copy(tab.at[srcb.at[s, GRP * jj + u]],
                                       rows.at[u], sem_g)
                      for u in range(GRP)]
                for d in dg:
                    d.wait()
                ds = [pltpu.async_copy(rows.at[u],
                                       acc.at[dstb.at[s, GRP * jj + u]],
                                       sem_s, add=True)
                      for u in range(GRP)]
                for d in ds:
                    d.wait()

        def pair(k, carry):
            block(2 * k, 0)
            block(2 * k + 1, 1)
            return carry

        lax.fori_loop(0, NB // 2, pair, 0)
        plsc.subcore_barrier()
        pltpu.sync_copy(acc.at[rsl], out.at[cid, p, rsl])
        plsc.subcore_barrier()


_hop = functools.partial(
    pl.kernel,
    out_type=jax.ShapeDtypeStruct((NC, 2, NP, HD), jnp.float32),
    mesh=_mesh,
    scratch_types=[
        pltpu.VMEM_SHARED((NP, HD), jnp.float32),
        pltpu.VMEM_SHARED((NP, HD), jnp.float32),
        pltpu.VMEM((2, BC, CH), jnp.int32),
        pltpu.VMEM((2, BC, CH), jnp.int32),
        pltpu.VMEM((GRP, CH, HD), jnp.float32),
        pltpu.SemaphoreType.DMA,
        pltpu.SemaphoreType.DMA,
    ],
)(_hop_body)


BLK = 1024


def _mm_body(x_ref, w_ref, o_ref):
    xx = jnp.concatenate([x_ref[0], x_ref[1]], axis=1)
    o_ref[0] = jnp.dot(xx, w_ref[0], preferred_element_type=jnp.float32)


_mm = pl.pallas_call(
    _mm_body,
    grid=(NP // BLK, 2),
    in_specs=[
        pl.BlockSpec((2, BLK, HD), lambda i, c: (0, i, 0)),
        pl.BlockSpec((1, D, HD), lambda i, c: (c, 0, 0)),
    ],
    out_specs=pl.BlockSpec((1, BLK, HD), lambda i, c: (c, i, 0)),
    out_shape=jax.ShapeDtypeStruct((2, NP, HD), jnp.float32),
)


def _comb_body(p_ref, o_ref):
    o_ref[0] = p_ref[0, 0] + p_ref[1, 0]


_comb = pl.pallas_call(
    _comb_body,
    grid=(NP // BLK, 2),
    in_specs=[pl.BlockSpec((NC, 1, BLK, HD), lambda i, c: (0, c, i, 0))],
    out_specs=pl.BlockSpec((1, BLK, HD), lambda i, c: (c, i, 0)),
    out_shape=jax.ShapeDtypeStruct((2, NP, HD), jnp.float32),
)


def _att_body(h0, h1, h2, h3, x_ref, w_ref, q_ref, b_ref, o_ref, *, relu):
    hs = [jnp.concatenate([h[0], h[1]], axis=1) for h in (h0, h1, h2, h3)]
    xx = jnp.concatenate([x_ref[0], x_ref[1]], axis=1)
    rs = [jnp.dot(h, q_ref[...], preferred_element_type=jnp.float32)
          for h in hs]
    t = rs[0][:, 0:1]
    ss = [t + r[:, 1:2] for r in rs]
    ss = [jnp.where(s >= 0, s, 0.2 * s) for s in ss]
    m = jnp.maximum(jnp.maximum(ss[0], ss[1]), jnp.maximum(ss[2], ss[3]))
    es = [jnp.exp(s - m) for s in ss]
    den = es[0] + es[1] + es[2] + es[3]
    o = (es[0] * hs[0] + es[1] * hs[1] + es[2] * hs[2] + es[3] * hs[3])
    o = o / den
    o = o + jnp.dot(xx, w_ref[...],
                    preferred_element_type=jnp.float32) + b_ref[0:1, :]
    if relu:
        o = jnp.maximum(o, 0.0)
    if o_ref.shape == (2, BLK, HD):
        o_ref[0] = o[:, :HD]
        o_ref[1] = o[:, HD:]
    else:
        o_ref[...] = o


def _att(relu, split_out):
    h_spec = pl.BlockSpec((2, BLK, HD), lambda i: (0, i, 0))
    if split_out:
        out_spec = pl.BlockSpec((2, BLK, HD), lambda i: (0, i, 0))
        out_shape = jax.ShapeDtypeStruct((2, NP, HD), jnp.float32)
    else:
        out_spec = pl.BlockSpec((BLK, D), lambda i: (i, 0))
        out_shape = jax.ShapeDtypeStruct((NP, D), jnp.float32)
    return pl.pallas_call(
        functools.partial(_att_body, relu=relu),
        grid=(NP // BLK,),
        in_specs=[
            h_spec, h_spec, h_spec, h_spec, h_spec,
            pl.BlockSpec((D, D), lambda i: (0, 0)),
            pl.BlockSpec((D, 8), lambda i: (0, 0)),
            pl.BlockSpec((1, D), lambda i: (0, 0)),
        ],
        out_specs=out_spec,
        out_shape=out_shape,
    )


_att1 = _att(True, True)
_att2 = _att(False, False)


def kernel(x, edge_index, W1, Wres1, b1, q1, W2, Wres2, b2, q2):
    src = edge_index[0].astype(jnp.int32)
    dst = edge_index[1].astype(jnp.int32)
    srcg = jnp.concatenate(
        [src, jnp.zeros((E_PAD - E,), jnp.int32)]).reshape(NW, T, CH)
    srcg = jnp.concatenate(
        [srcg, jnp.zeros((NW, T2 - T, CH), jnp.int32)],
        axis=1).reshape(NW * T2, CH)
    dstg = jnp.concatenate(
        [dst, jnp.full((E_PAD - E,), TRASH, jnp.int32)]).reshape(NW, T, CH)
    dstg = jnp.concatenate(
        [dstg, jnp.full((NW, T2 - T, CH), TRASH, jnp.int32)],
        axis=1).reshape(NW * T2, CH)
    xp = jnp.zeros((NP, D), jnp.float32).at[:N].set(x)
    xs = jnp.stack([xp[:, :HD], xp[:, HD:]])
    zer = jnp.zeros((NP, HD), jnp.float32)

    def layer(xin, W, Wres, b, q, last):
        Ws = jnp.stack([W[:, :HD], W[:, HD:]])
        h0 = _mm(xin, Ws)
        cur = h0
        hs = [h0]
        for _ in range(K):
            p = _hop(cur, srcg, dstg, zer)
            cur = _comb(p)
            hs.append(cur)
        att = _att2 if last else _att1
        qm = jnp.zeros((D, 8), jnp.float32)
        qm = qm.at[:, 0].set(q[:D]).at[:, 1].set(q[D:])
        b2d = b.reshape(1, D)
        return att(hs[0], hs[1], hs[2], hs[3], xin, Wres, qm, b2d)

    h = layer(xs, W1, Wres1, b1, q1, False)
    out = layer(h, W2, Wres2, b2, q2, True)
    return out[:N]


# speedup vs baseline: 6.3029x; 1.1657x over previous
import functools

import jax
import jax.numpy as jnp
from jax import lax
from jax.experimental import pallas as pl
from jax.experimental.pallas import tpu as pltpu
from jax.experimental.pallas import tpu_sc as plsc

N = 10000
D = 128
HD = D // 2
E = 320000
K = 3

NC = 2
NS = 16
NW = NC * NS

CH = 128
GRP = 4
BC = 8
T = 80
T2 = T + 16
NB = T // BC
E_PAD = NW * T * CH
NP = 10240
RPT = NP // NS
TRASH = N

_mesh = plsc.VectorSubcoreMesh(core_axis_name="c", subcore_axis_name="s")


def _hop_body(cur2, srcg, dstg, zer, out, tab, acc, srcb, dstb, rows,
              sem_g, sem_s):
    cid = lax.axis_index("c")
    sid = lax.axis_index("s")
    wid = sid * NC + cid
    row0 = wid * T2
    rsl = pl.ds(sid * RPT, RPT)

    for p in range(2):
        pltpu.sync_copy(cur2.at[p, rsl], tab.at[rsl])
        pltpu.sync_copy(zer.at[rsl], acc.at[rsl])
        plsc.subcore_barrier()

        def block(b, s):
            pltpu.sync_copy(srcg.at[pl.ds(row0 + b * BC, BC)], srcb.at[s])
            pltpu.sync_copy(dstg.at[pl.ds(row0 + b * BC, BC)], dstb.at[s])
            for jj in range(BC // GRP):
                dg = [pltpu.async_copy(tab.at[srcb.at[s, GRP * jj + u]],
                                       rows.at[u], sem_g)
                      for u in range(GRP)]
                for d in dg:
                    d.wait()
                ds = [pltpu.async_copy(rows.at[u],
                                       acc.at[dstb.at[s, GRP * jj + u]],
                                       sem_s, add=True)
                      for u in range(GRP)]
                for d in ds:
                    d.wait()

        def pair(k, carry):
            block(2 * k, 0)
            block(2 * k + 1, 1)
            return carry

        lax.fori_loop(0, NB // 2, pair, 0)
        plsc.subcore_barrier()
        pltpu.sync_copy(acc.at[rsl], out.at[cid, p, rsl])
        plsc.subcore_barrier()


_hop = functools.partial(
    pl.kernel,
    out_type=jax.ShapeDtypeStruct((NC, 2, NP, HD), jnp.float32),
    mesh=_mesh,
    scratch_types=[
        pltpu.VMEM_SHARED((NP, HD), jnp.float32),
        pltpu.VMEM_SHARED((NP, HD), jnp.float32),
        pltpu.VMEM((2, BC, CH), jnp.int32),
        pltpu.VMEM((2, BC, CH), jnp.int32),
        pltpu.VMEM((GRP, CH, HD), jnp.float32),
        pltpu.SemaphoreType.DMA,
        pltpu.SemaphoreType.DMA,
    ],
    compiler_params=pltpu.CompilerParams(use_tc_tiling_on_sc=False),
)(_hop_body)


BLK = 1024


def _mm_body(x_ref, w_ref, o_ref):
    xx = jnp.concatenate([x_ref[0], x_ref[1]], axis=1)
    o_ref[0] = jnp.dot(xx, w_ref[0], preferred_element_type=jnp.float32)


_mm = pl.pallas_call(
    _mm_body,
    grid=(NP // BLK, 2),
    in_specs=[
        pl.BlockSpec((2, BLK, HD), lambda i, c: (0, i, 0)),
        pl.BlockSpec((1, D, HD), lambda i, c: (c, 0, 0)),
    ],
    out_specs=pl.BlockSpec((1, BLK, HD), lambda i, c: (c, i, 0)),
    out_shape=jax.ShapeDtypeStruct((2, NP, HD), jnp.float32),
)


def _comb_body(p_ref, o_ref):
    o_ref[0] = p_ref[0, 0] + p_ref[1, 0]


_comb = pl.pallas_call(
    _comb_body,
    grid=(NP // BLK, 2),
    in_specs=[pl.BlockSpec((NC, 1, BLK, HD), lambda i, c: (0, c, i, 0))],
    out_specs=pl.BlockSpec((1, BLK, HD), lambda i, c: (c, i, 0)),
    out_shape=jax.ShapeDtypeStruct((2, NP, HD), jnp.float32),
)


def _att_body(h0, h1, h2, h3, x_ref, w_ref, q_ref, b_ref, o_ref, *, relu):
    hs = [jnp.concatenate([h[0], h[1]], axis=1) for h in (h0, h1, h2, h3)]
    xx = jnp.concatenate([x_ref[0], x_ref[1]], axis=1)
    rs = [jnp.dot(h, q_ref[...], preferred_element_type=jnp.float32)
          for h in hs]
    t = rs[0][:, 0:1]
    ss = [t + r[:, 1:2] for r in rs]
    ss = [jnp.where(s >= 0, s, 0.2 * s) for s in ss]
    m = jnp.maximum(jnp.maximum(ss[0], ss[1]), jnp.maximum(ss[2], ss[3]))
    es = [jnp.exp(s - m) for s in ss]
    den = es[0] + es[1] + es[2] + es[3]
    o = (es[0] * hs[0] + es[1] * hs[1] + es[2] * hs[2] + es[3] * hs[3])
    o = o / den
    o = o + jnp.dot(xx, w_ref[...],
                    preferred_element_type=jnp.float32) + b_ref[0:1, :]
    if relu:
        o = jnp.maximum(o, 0.0)
    if o_ref.shape == (2, BLK, HD):
        o_ref[0] = o[:, :HD]
        o_ref[1] = o[:, HD:]
    else:
        o_ref[...] = o


def _att(relu, split_out):
    h_spec = pl.BlockSpec((2, BLK, HD), lambda i: (0, i, 0))
    if split_out:
        out_spec = pl.BlockSpec((2, BLK, HD), lambda i: (0, i, 0))
        out_shape = jax.ShapeDtypeStruct((2, NP, HD), jnp.float32)
    else:
        out_spec = pl.BlockSpec((BLK, D), lambda i: (i, 0))
        out_shape = jax.ShapeDtypeStruct((NP, D), jnp.float32)
    return pl.pallas_call(
        functools.partial(_att_body, relu=relu),
        grid=(NP // BLK,),
        in_specs=[
            h_spec, h_spec, h_spec, h_spec, h_spec,
            pl.BlockSpec((D, D), lambda i: (0, 0)),
            pl.BlockSpec((D, 8), lambda i: (0, 0)),
            pl.BlockSpec((1, D), lambda i: (0, 0)),
        ],
        out_specs=out_spec,
        out_shape=out_shape,
    )


_att1 = _att(True, True)
_att2 = _att(False, False)


def kernel(x, edge_index, W1, Wres1, b1, q1, W2, Wres2, b2, q2):
    src = edge_index[0].astype(jnp.int32)
    dst = edge_index[1].astype(jnp.int32)
    srcg = jnp.concatenate(
        [src, jnp.zeros((E_PAD - E,), jnp.int32)]).reshape(NW, T, CH)
    srcg = jnp.concatenate(
        [srcg, jnp.zeros((NW, T2 - T, CH), jnp.int32)],
        axis=1).reshape(NW * T2, CH)
    dstg = jnp.concatenate(
        [dst, jnp.full((E_PAD - E,), TRASH, jnp.int32)]).reshape(NW, T, CH)
    dstg = jnp.concatenate(
        [dstg, jnp.full((NW, T2 - T, CH), TRASH, jnp.int32)],
        axis=1).reshape(NW * T2, CH)
    xp = jnp.zeros((NP, D), jnp.float32).at[:N].set(x)
    xs = jnp.stack([xp[:, :HD], xp[:, HD:]])
    zer = jnp.zeros((NP, HD), jnp.float32)

    def layer(xin, W, Wres, b, q, last):
        Ws = jnp.stack([W[:, :HD], W[:, HD:]])
        h0 = _mm(xin, Ws)
        cur = h0
        hs = [h0]
        for _ in range(K):
            p = _hop(cur, srcg, dstg, zer)
            cur = _comb(p)
            hs.append(cur)
        att = _att2 if last else _att1
        qm = jnp.zeros((D, 8), jnp.float32)
        qm = qm.at[:, 0].set(q[:D]).at[:, 1].set(q[D:])
        b2d = b.reshape(1, D)
        return att(hs[0], hs[1], hs[2], hs[3], xin, Wres, qm, b2d)

    h = layer(xs, W1, Wres1, b1, q1, False)
    out = layer(h, W2, Wres2, b2, q2, True)
    return out[:N]
